# baseline (speedup 1.0000x reference)
import functools

import jax
import jax.numpy as jnp
from jax import lax
from jax.experimental import pallas as pl
from jax.experimental.pallas import tpu as pltpu
from jax.experimental.pallas import tpu_sc as plsc

B = 16384
HIST = 50
D = 16
ITEM_ROWS = 1000000

NC = 2
NS = 16
NW = NC * NS
BPW = B // NW
CB = 64
NCHUNK = BPW // CB
HIST_IDX = CB * HIST
GSZ = 128
NGATHER = HIST_IDX // GSZ


def _sc_gather_body(idx_u_hbm, idx_p_hbm, idx_c_hbm, idx_i_hbm, idx_h_hbm,
                    user_t1, prov_t1, city_t1, item_t1,
                    out_u, out_p, out_c, out_i, out_h,
                    iu_v, ip_v, ic_v, ii_v, ih_v,
                    ru_v, rp_v, rc_v, ri_v, rh_v, hs_v, sem):
    user_t, prov_t, city_t, item_t = user_t1, prov_t1, city_t1, item_t1
    wid = lax.axis_index("s") * NC + lax.axis_index("c")

    def chunk_body(c, carry):
        base = wid * BPW + c * CB
        pltpu.sync_copy(idx_u_hbm.at[pl.ds(base, CB)], iu_v)
        pltpu.sync_copy(idx_p_hbm.at[pl.ds(base, CB)], ip_v)
        pltpu.sync_copy(idx_c_hbm.at[pl.ds(base, CB)], ic_v)
        pltpu.sync_copy(idx_i_hbm.at[pl.ds(base, CB)], ii_v)
        pltpu.sync_copy(idx_h_hbm.at[pl.ds(base * HIST, HIST_IDX)], ih_v)

        cps = [
            pltpu.async_copy(user_t.at[iu_v], ru_v, sem),
            pltpu.async_copy(prov_t.at[ip_v], rp_v, sem),
            pltpu.async_copy(city_t.at[ic_v], rc_v, sem),
            pltpu.async_copy(item_t.at[ii_v], ri_v, sem),
        ]
        for k in range(NGATHER):
            cps.append(pltpu.async_copy(
                item_t.at[ih_v.at[pl.ds(k * GSZ, GSZ)]],
                rh_v.at[pl.ds(k * GSZ, GSZ)], sem))
        for cp in cps:
            cp.wait()

        def row_body(b, carry2):
            acc = rh_v[b * HIST, :]
            for j in range(1, HIST):
                acc = acc + rh_v[b * HIST + j, :]
            hs_v[b, :] = acc
            return carry2
        lax.fori_loop(0, CB, row_body, 0, unroll=False)

        pltpu.sync_copy(ru_v, out_u.at[pl.ds(base, CB)])
        pltpu.sync_copy(rp_v, out_p.at[pl.ds(base, CB)])
        pltpu.sync_copy(rc_v, out_c.at[pl.ds(base, CB)])
        pltpu.sync_copy(ri_v, out_i.at[pl.ds(base, CB)])
        pltpu.sync_copy(hs_v, out_h.at[pl.ds(base, CB)])
        return carry

    lax.fori_loop(0, NCHUNK, chunk_body, 0, unroll=False)


@functools.cache
def _sc_gather():
    return pl.kernel(
        _sc_gather_body,
        out_type=[jax.ShapeDtypeStruct((B, D), jnp.float32)] * 5,
        mesh=plsc.VectorSubcoreMesh(core_axis_name="c", subcore_axis_name="s",
                                    num_cores=NC, num_subcores=NS),
        compiler_params=pltpu.CompilerParams(use_tc_tiling_on_sc=False),
        scratch_types=[
        pltpu.VMEM((CB,), jnp.int32),
        pltpu.VMEM((CB,), jnp.int32),
        pltpu.VMEM((CB,), jnp.int32),
        pltpu.VMEM((CB,), jnp.int32),
        pltpu.VMEM((HIST_IDX,), jnp.int32),
        pltpu.VMEM((CB, D), jnp.float32),
        pltpu.VMEM((CB, D), jnp.float32),
        pltpu.VMEM((CB, D), jnp.float32),
        pltpu.VMEM((CB, D), jnp.float32),
        pltpu.VMEM((HIST_IDX, D), jnp.float32),
            pltpu.VMEM((CB, D), jnp.float32),
            pltpu.SemaphoreType.DMA,
        ],
    )


RB = 2048


def _tc_repack_body(u_ref, p_ref, c_ref, i_ref, ou_ref, op_ref, oc_ref, oi_ref):
    ou_ref[:] = u_ref[:].T
    op_ref[:] = p_ref[:].T
    oc_ref[:] = c_ref[:].T
    oi_ref[:] = i_ref[:].T


def _tc_repack(user_tt, prov_tt, city_tt, item_tt):
    grid = (pl.cdiv(ITEM_ROWS + 1, RB),)
    col = pl.BlockSpec((D, RB), lambda g: (0, g))
    rowb = pl.BlockSpec((RB, D), lambda g: (g, 0))
    return pl.pallas_call(
        _tc_repack_body,
        grid=grid,
        in_specs=[col, col, col, col],
        out_specs=[rowb, rowb, rowb, rowb],
        out_shape=[jax.ShapeDtypeStruct((ITEM_ROWS, D), jnp.float32)] * 4,
    )(user_tt, prov_tt, city_tt, item_tt)


BM = 2048


def _tc_mlp_body(u_ref, p_ref, c_ref, i_ref, h_ref, x_ref,
                 w1u, w1p, w1c, w1h, w1x, b1_ref,
                 w2_ref, b2_ref, w3_ref, b3_ref, o_ref):
    f32 = jnp.float32
    h1 = (jnp.dot(u_ref[:], w1u[:], preferred_element_type=f32)
          + jnp.dot(p_ref[:], w1p[:], preferred_element_type=f32)
          + jnp.dot(c_ref[:], w1c[:], preferred_element_type=f32)
          + jnp.dot(h_ref[:] * (1.0 / HIST), w1h[:], preferred_element_type=f32)
          + jnp.dot(x_ref[:], w1x[:], preferred_element_type=f32)
          + b1_ref[:])
    h2 = jnp.dot(h1, w2_ref[:], preferred_element_type=f32) + b2_ref[:]
    u = jnp.dot(h2, w3_ref[:], preferred_element_type=f32) + b3_ref[:]
    logits = jnp.sum(u * i_ref[:], axis=1, keepdims=True)
    o_ref[:] = jax.nn.sigmoid(logits)


def _tc_mlp(user_e, prov_e, city_e, item_e, hist_s, cont,
            w1u, w1p, w1c, w1h, w1x, b1, w2, b2, w3, b3):
    row = lambda bm, w: pl.BlockSpec((bm, w), lambda i: (i, 0))
    full = lambda s: pl.BlockSpec(s, lambda i: (0, 0))
    return pl.pallas_call(
        _tc_mlp_body,
        grid=(B // BM,),
        in_specs=[
            row(BM, D), row(BM, D), row(BM, D), row(BM, D), row(BM, D),
            row(BM, 19),
            full((D, 128)), full((D, 128)), full((D, 128)), full((D, 128)),
            full((19, 128)), full((1, 128)),
            full((128, 32)), full((1, 32)),
            full((32, 16)), full((1, 16)),
        ],
        out_specs=pl.BlockSpec((BM, 1), lambda i: (i, 0)),
        out_shape=jax.ShapeDtypeStruct((B, 1), jnp.float32),
    )(user_e, prov_e, city_e, item_e, hist_s, cont,
      w1u, w1p, w1c, w1h, w1x, b1, w2, b2, w3, b3)


def kernel(disc, cont, itemId, item_table, user_table, city_table, prov_table,
           W1, b1, W2, b2, W3, b3):
    idx_u = disc[:, 0]
    idx_p = disc[:, 1]
    idx_c = disc[:, 2]
    idx_h = disc[:, 3:].reshape(-1)

    user_r, prov_r, city_r, item_r = _tc_repack(
        user_table.T, prov_table.T, city_table.T, item_table.T)

    user_e, prov_e, city_e, item_e, hist_s = _sc_gather()(
        idx_u, idx_p, idx_c, itemId, idx_h,
        user_r, prov_r, city_r, item_r)

    out = _tc_mlp(
        user_e, prov_e, city_e, item_e, hist_s, cont,
        W1[0:16], W1[16:32], W1[32:48], W1[48:64], W1[64:83],
        b1.reshape(1, 128), W2, b2.reshape(1, 32), W3, b3.reshape(1, 16))
    return out.reshape(B)


# speedup vs baseline: 1.0872x; 1.0872x over previous
import functools

import jax
import jax.numpy as jnp
from jax import lax
from jax.experimental import pallas as pl
from jax.experimental.pallas import tpu as pltpu
from jax.experimental.pallas import tpu_sc as plsc

B = 16384
HIST = 50
D = 16
ITEM_ROWS = 1000000

NC = 2
NS = 16
NW = NC * NS
BPW = B // NW
CB = 64
NCHUNK = BPW // CB
HIST_IDX = CB * HIST
GSZ = 128
NGATHER = HIST_IDX // GSZ


def _sc_gather_body(idx_u_hbm, idx_p_hbm, idx_c_hbm, idx_i_hbm, idx_h_hbm,
                    user_t1, prov_t1, city_t1, item_t1,
                    out_u, out_p, out_c, out_i, out_h,
                    iu_v, ip_v, ic_v, ii_v, ih_v,
                    ru_v, rp_v, rc_v, ri_v, rh_v, hs_v, sem):
    user_t, prov_t, city_t, item_t = user_t1, prov_t1, city_t1, item_t1
    wid = lax.axis_index("s") * NC + lax.axis_index("c")

    def chunk_body(c, carry):
        base = wid * BPW + c * CB
        pltpu.sync_copy(idx_u_hbm.at[pl.ds(base, CB)], iu_v)
        pltpu.sync_copy(idx_p_hbm.at[pl.ds(base, CB)], ip_v)
        pltpu.sync_copy(idx_c_hbm.at[pl.ds(base, CB)], ic_v)
        pltpu.sync_copy(idx_i_hbm.at[pl.ds(base, CB)], ii_v)
        pltpu.sync_copy(idx_h_hbm.at[pl.ds(base * HIST, HIST_IDX)], ih_v)

        cps = [
            pltpu.async_copy(user_t.at[iu_v], ru_v, sem),
            pltpu.async_copy(prov_t.at[ip_v], rp_v, sem),
            pltpu.async_copy(city_t.at[ic_v], rc_v, sem),
            pltpu.async_copy(item_t.at[ii_v], ri_v, sem),
        ]
        for k in range(NGATHER):
            cps.append(pltpu.async_copy(
                item_t.at[ih_v.at[pl.ds(k * GSZ, GSZ)]],
                rh_v.at[pl.ds(k * GSZ, GSZ)], sem))
        for cp in cps:
            cp.wait()

        def row_body(b, carry2):
            acc = rh_v[b * HIST, :]
            for j in range(1, HIST):
                acc = acc + rh_v[b * HIST + j, :]
            hs_v[b, :] = acc
            return carry2
        lax.fori_loop(0, CB, row_body, 0, unroll=False)

        pltpu.sync_copy(ru_v, out_u.at[pl.ds(base, CB)])
        pltpu.sync_copy(rp_v, out_p.at[pl.ds(base, CB)])
        pltpu.sync_copy(rc_v, out_c.at[pl.ds(base, CB)])
        pltpu.sync_copy(ri_v, out_i.at[pl.ds(base, CB)])
        pltpu.sync_copy(hs_v, out_h.at[pl.ds(base, CB)])
        return carry

    lax.fori_loop(0, NCHUNK, chunk_body, 0, unroll=False)


@functools.cache
def _sc_gather():
    return pl.kernel(
        _sc_gather_body,
        out_type=[jax.ShapeDtypeStruct((B, D), jnp.float32)] * 5,
        mesh=plsc.VectorSubcoreMesh(core_axis_name="c", subcore_axis_name="s",
                                    num_cores=NC, num_subcores=NS),
        compiler_params=pltpu.CompilerParams(use_tc_tiling_on_sc=False),
        scratch_types=[
        pltpu.VMEM((CB,), jnp.int32),
        pltpu.VMEM((CB,), jnp.int32),
        pltpu.VMEM((CB,), jnp.int32),
        pltpu.VMEM((CB,), jnp.int32),
        pltpu.VMEM((HIST_IDX,), jnp.int32),
        pltpu.VMEM((CB, D), jnp.float32),
        pltpu.VMEM((CB, D), jnp.float32),
        pltpu.VMEM((CB, D), jnp.float32),
        pltpu.VMEM((CB, D), jnp.float32),
        pltpu.VMEM((HIST_IDX, D), jnp.float32),
            pltpu.VMEM((CB, D), jnp.float32),
            pltpu.SemaphoreType.DMA,
        ],
    )


RB = 8192


def _tc_repack_body(u_ref, p_ref, c_ref, i_ref, ou_ref, op_ref, oc_ref, oi_ref):
    eye = (jax.lax.broadcasted_iota(jnp.int32, (D, D), 0)
           == jax.lax.broadcasted_iota(jnp.int32, (D, D), 1)).astype(jnp.float32)
    t = lambda x: jax.lax.dot_general(x, eye, (((0,), (0,)), ((), ())),
                                      preferred_element_type=jnp.float32)
    ou_ref[:] = t(u_ref[:])
    op_ref[:] = t(p_ref[:])
    oc_ref[:] = t(c_ref[:])
    oi_ref[:] = t(i_ref[:])


def _tc_repack(user_tt, prov_tt, city_tt, item_tt):
    grid = (pl.cdiv(ITEM_ROWS + 1, RB),)
    col = pl.BlockSpec((D, RB), lambda g: (0, g))
    rowb = pl.BlockSpec((RB, D), lambda g: (g, 0))
    return pl.pallas_call(
        _tc_repack_body,
        grid=grid,
        in_specs=[col, col, col, col],
        out_specs=[rowb, rowb, rowb, rowb],
        out_shape=[jax.ShapeDtypeStruct((ITEM_ROWS, D), jnp.float32)] * 4,
    )(user_tt, prov_tt, city_tt, item_tt)


BM = 2048


def _tc_mlp_body(u_ref, p_ref, c_ref, i_ref, h_ref, x_ref,
                 w1u, w1p, w1c, w1h, w1x, b1_ref,
                 w2_ref, b2_ref, w3_ref, b3_ref, o_ref):
    f32 = jnp.float32
    h1 = (jnp.dot(u_ref[:], w1u[:], preferred_element_type=f32)
          + jnp.dot(p_ref[:], w1p[:], preferred_element_type=f32)
          + jnp.dot(c_ref[:], w1c[:], preferred_element_type=f32)
          + jnp.dot(h_ref[:] * (1.0 / HIST), w1h[:], preferred_element_type=f32)
          + jnp.dot(x_ref[:], w1x[:], preferred_element_type=f32)
          + b1_ref[:])
    h2 = jnp.dot(h1, w2_ref[:], preferred_element_type=f32) + b2_ref[:]
    u = jnp.dot(h2, w3_ref[:], preferred_element_type=f32) + b3_ref[:]
    logits = jnp.sum(u * i_ref[:], axis=1, keepdims=True)
    o_ref[:] = jax.nn.sigmoid(logits)


def _tc_mlp(user_e, prov_e, city_e, item_e, hist_s, cont,
            w1u, w1p, w1c, w1h, w1x, b1, w2, b2, w3, b3):
    row = lambda bm, w: pl.BlockSpec((bm, w), lambda i: (i, 0))
    full = lambda s: pl.BlockSpec(s, lambda i: (0, 0))
    return pl.pallas_call(
        _tc_mlp_body,
        grid=(B // BM,),
        in_specs=[
            row(BM, D), row(BM, D), row(BM, D), row(BM, D), row(BM, D),
            row(BM, 19),
            full((D, 128)), full((D, 128)), full((D, 128)), full((D, 128)),
            full((19, 128)), full((1, 128)),
            full((128, 32)), full((1, 32)),
            full((32, 16)), full((1, 16)),
        ],
        out_specs=pl.BlockSpec((BM, 1), lambda i: (i, 0)),
        out_shape=jax.ShapeDtypeStruct((B, 1), jnp.float32),
    )(user_e, prov_e, city_e, item_e, hist_s, cont,
      w1u, w1p, w1c, w1h, w1x, b1, w2, b2, w3, b3)


def kernel(disc, cont, itemId, item_table, user_table, city_table, prov_table,
           W1, b1, W2, b2, W3, b3):
    idx_u = disc[:, 0]
    idx_p = disc[:, 1]
    idx_c = disc[:, 2]
    idx_h = disc[:, 3:].reshape(-1)

    user_r, prov_r, city_r, item_r = _tc_repack(
        user_table.T, prov_table.T, city_table.T, item_table.T)

    user_e, prov_e, city_e, item_e, hist_s = _sc_gather()(
        idx_u, idx_p, idx_c, itemId, idx_h,
        user_r, prov_r, city_r, item_r)

    out = _tc_mlp(
        user_e, prov_e, city_e, item_e, hist_s, cont,
        W1[0:16], W1[16:32], W1[32:48], W1[48:64], W1[64:83],
        b1.reshape(1, 128), W2, b2.reshape(1, 32), W3, b3.reshape(1, 16))
    return out.reshape(B)


# speedup vs baseline: 1.5059x; 1.3850x over previous
import functools

import jax
import jax.numpy as jnp
from jax import lax
from jax.experimental import pallas as pl
from jax.experimental.pallas import tpu as pltpu
from jax.experimental.pallas import tpu_sc as plsc

B = 16384
HIST = 50
D = 16
ITEM_ROWS = 1000000

NC = 2
NS = 16
NW = NC * NS
BPW = B // NW
CB = 64
NCHUNK = BPW // CB
HIST_IDX = CB * HIST
GSZ = 128
NGATHER = HIST_IDX // GSZ


def _sc_gather_body(idx_u_hbm, idx_p_hbm, idx_c_hbm, idx_i_hbm, idx_h_hbm,
                    tab4,
                    out_u, out_p, out_c, out_i, out_h,
                    iu_v, ip_v, ic_v, ii_v, ih_v,
                    ru_v, rp_v, rc_v, ri_v, rh_v, hs_v, sem):
    wid = lax.axis_index("s") * NC + lax.axis_index("c")

    def chunk_body(c, carry):
        base = wid * BPW + c * CB
        pltpu.sync_copy(idx_u_hbm.at[pl.ds(base, CB)], iu_v)
        pltpu.sync_copy(idx_p_hbm.at[pl.ds(base, CB)], ip_v)
        pltpu.sync_copy(idx_c_hbm.at[pl.ds(base, CB)], ic_v)
        pltpu.sync_copy(idx_i_hbm.at[pl.ds(base, CB)], ii_v)
        pltpu.sync_copy(idx_h_hbm.at[pl.ds(base * HIST, HIST_IDX)], ih_v)

        for s_ in range(CB // 16):
            sl = pl.ds(s_ * 16, 16)
            iu_v[sl] = iu_v[sl] * 4
            ip_v[sl] = ip_v[sl] * 4 + 1
            ic_v[sl] = ic_v[sl] * 4 + 2
            ii_v[sl] = ii_v[sl] * 4 + 3

        def hidx_body(s_, carry2):
            sl = pl.ds(s_ * 16, 16)
            ih_v[sl] = ih_v[sl] * 4 + 3
            return carry2
        lax.fori_loop(0, HIST_IDX // 16, hidx_body, 0, unroll=4)

        cps = [
            pltpu.async_copy(tab4.at[iu_v], ru_v, sem),
            pltpu.async_copy(tab4.at[ip_v], rp_v, sem),
            pltpu.async_copy(tab4.at[ic_v], rc_v, sem),
            pltpu.async_copy(tab4.at[ii_v], ri_v, sem),
        ]
        for k in range(NGATHER):
            cps.append(pltpu.async_copy(
                tab4.at[ih_v.at[pl.ds(k * GSZ, GSZ)]],
                rh_v.at[pl.ds(k * GSZ, GSZ)], sem))
        for cp in cps:
            cp.wait()

        def row_body(b, carry2):
            acc = rh_v[b * HIST, :]
            for j in range(1, HIST):
                acc = acc + rh_v[b * HIST + j, :]
            hs_v[b, :] = acc
            return carry2
        lax.fori_loop(0, CB, row_body, 0, unroll=False)

        pltpu.sync_copy(ru_v, out_u.at[pl.ds(base, CB)])
        pltpu.sync_copy(rp_v, out_p.at[pl.ds(base, CB)])
        pltpu.sync_copy(rc_v, out_c.at[pl.ds(base, CB)])
        pltpu.sync_copy(ri_v, out_i.at[pl.ds(base, CB)])
        pltpu.sync_copy(hs_v, out_h.at[pl.ds(base, CB)])
        return carry

    lax.fori_loop(0, NCHUNK, chunk_body, 0, unroll=False)


@functools.cache
def _sc_gather():
    return pl.kernel(
        _sc_gather_body,
        out_type=[jax.ShapeDtypeStruct((B, D), jnp.float32)] * 5,
        mesh=plsc.VectorSubcoreMesh(core_axis_name="c", subcore_axis_name="s",
                                    num_cores=NC, num_subcores=NS),
        compiler_params=pltpu.CompilerParams(use_tc_tiling_on_sc=False),
        scratch_types=[
        pltpu.VMEM((CB,), jnp.int32),
        pltpu.VMEM((CB,), jnp.int32),
        pltpu.VMEM((CB,), jnp.int32),
        pltpu.VMEM((CB,), jnp.int32),
        pltpu.VMEM((HIST_IDX,), jnp.int32),
        pltpu.VMEM((CB, D), jnp.float32),
        pltpu.VMEM((CB, D), jnp.float32),
        pltpu.VMEM((CB, D), jnp.float32),
        pltpu.VMEM((CB, D), jnp.float32),
        pltpu.VMEM((HIST_IDX, D), jnp.float32),
            pltpu.VMEM((CB, D), jnp.float32),
            pltpu.SemaphoreType.DMA,
        ],
    )


RB = 8192


def _tc_repack_body(u_ref, p_ref, c_ref, i_ref, o_ref):
    eye = (jax.lax.broadcasted_iota(jnp.int32, (D, D), 0)
           == jax.lax.broadcasted_iota(jnp.int32, (D, D), 1)).astype(jnp.float32)
    t = lambda x: jax.lax.dot_general(x, eye, (((0,), (0,)), ((), ())),
                                      preferred_element_type=jnp.float32)
    o_ref[:] = jnp.concatenate(
        [t(u_ref[:]), t(p_ref[:]), t(c_ref[:]), t(i_ref[:])], axis=1)


def _tc_repack(user_tt, prov_tt, city_tt, item_tt):
    grid = (pl.cdiv(ITEM_ROWS + 1, RB),)
    col = pl.BlockSpec((D, RB), lambda g: (0, g))
    rowb = pl.BlockSpec((RB, 4 * D), lambda g: (g, 0))
    return pl.pallas_call(
        _tc_repack_body,
        grid=grid,
        in_specs=[col, col, col, col],
        out_specs=rowb,
        out_shape=jax.ShapeDtypeStruct((ITEM_ROWS, 4 * D), jnp.float32),
    )(user_tt, prov_tt, city_tt, item_tt)


BM = 2048


def _tc_mlp_body(u_ref, p_ref, c_ref, i_ref, h_ref, x_ref,
                 w1u, w1p, w1c, w1h, w1x, b1_ref,
                 w2_ref, b2_ref, w3_ref, b3_ref, o_ref):
    f32 = jnp.float32
    h1 = (jnp.dot(u_ref[:], w1u[:], preferred_element_type=f32)
          + jnp.dot(p_ref[:], w1p[:], preferred_element_type=f32)
          + jnp.dot(c_ref[:], w1c[:], preferred_element_type=f32)
          + jnp.dot(h_ref[:] * (1.0 / HIST), w1h[:], preferred_element_type=f32)
          + jnp.dot(x_ref[:], w1x[:], preferred_element_type=f32)
          + b1_ref[:])
    h2 = jnp.dot(h1, w2_ref[:], preferred_element_type=f32) + b2_ref[:]
    u = jnp.dot(h2, w3_ref[:], preferred_element_type=f32) + b3_ref[:]
    logits = jnp.sum(u * i_ref[:], axis=1, keepdims=True)
    o_ref[:] = jax.nn.sigmoid(logits)


def _tc_mlp(user_e, prov_e, city_e, item_e, hist_s, cont,
            w1u, w1p, w1c, w1h, w1x, b1, w2, b2, w3, b3):
    row = lambda bm, w: pl.BlockSpec((bm, w), lambda i: (i, 0))
    full = lambda s: pl.BlockSpec(s, lambda i: (0, 0))
    return pl.pallas_call(
        _tc_mlp_body,
        grid=(B // BM,),
        in_specs=[
            row(BM, D), row(BM, D), row(BM, D), row(BM, D), row(BM, D),
            row(BM, 19),
            full((D, 128)), full((D, 128)), full((D, 128)), full((D, 128)),
            full((19, 128)), full((1, 128)),
            full((128, 32)), full((1, 32)),
            full((32, 16)), full((1, 16)),
        ],
        out_specs=pl.BlockSpec((BM, 1), lambda i: (i, 0)),
        out_shape=jax.ShapeDtypeStruct((B, 1), jnp.float32),
    )(user_e, prov_e, city_e, item_e, hist_s, cont,
      w1u, w1p, w1c, w1h, w1x, b1, w2, b2, w3, b3)


def kernel(disc, cont, itemId, item_table, user_table, city_table, prov_table,
           W1, b1, W2, b2, W3, b3):
    idx_u = disc[:, 0]
    idx_p = disc[:, 1]
    idx_c = disc[:, 2]
    idx_h = disc[:, 3:].reshape(-1)

    packed = _tc_repack(
        user_table.T, prov_table.T, city_table.T, item_table.T)
    packed4 = packed.reshape(4 * ITEM_ROWS, D)

    user_e, prov_e, city_e, item_e, hist_s = _sc_gather()(
        idx_u, idx_p, idx_c, itemId, idx_h, packed4)

    out = _tc_mlp(
        user_e, prov_e, city_e, item_e, hist_s, cont,
        W1[0:16], W1[16:32], W1[32:48], W1[48:64], W1[64:83],
        b1.reshape(1, 128), W2, b2.reshape(1, 32), W3, b3.reshape(1, 16))
    return out.reshape(B)


# speedup vs baseline: 2.7144x; 1.8026x over previous
import functools

import jax
import jax.numpy as jnp
from jax import lax
from jax.experimental import pallas as pl
from jax.experimental.pallas import tpu as pltpu
from jax.experimental.pallas import tpu_sc as plsc

B = 16384
HIST = 50
D = 16
ITEM_ROWS = 1000000

NC = 2
NS = 16
NW = NC * NS
BPW = B // NW
CB = 64
NCHUNK = BPW // CB
HIST_IDX = CB * HIST
GSZ = 128
NGATHER = HIST_IDX // GSZ


def _sc_gather_body(idx_u_hbm, idx_p_hbm, idx_c_hbm, idx_i_hbm, idx_h_hbm,
                    tab4,
                    out_u, out_p, out_c, out_i, out_h,
                    iu_v, ip_v, ic_v, ii_v, ih_v,
                    ru_v, rp_v, rc_v, ri_v, rh_v, hs_v, sem):
    wid = lax.axis_index("s") * NC + lax.axis_index("c")

    def chunk_body(c, carry):
        base = wid * BPW + c * CB
        pltpu.sync_copy(idx_u_hbm.at[pl.ds(base, CB)], iu_v)
        pltpu.sync_copy(idx_p_hbm.at[pl.ds(base, CB)], ip_v)
        pltpu.sync_copy(idx_c_hbm.at[pl.ds(base, CB)], ic_v)
        pltpu.sync_copy(idx_i_hbm.at[pl.ds(base, CB)], ii_v)
        pltpu.sync_copy(idx_h_hbm.at[pl.ds(base * HIST, HIST_IDX)], ih_v)

        for s_ in range(CB // 16):
            sl = pl.ds(s_ * 16, 16)
            iu_v[sl] = iu_v[sl] * 4
            ip_v[sl] = ip_v[sl] * 4 + 1
            ic_v[sl] = ic_v[sl] * 4 + 2
            ii_v[sl] = ii_v[sl] * 4 + 3

        def hidx_body(s_, carry2):
            sl = pl.ds(s_ * 16, 16)
            ih_v[sl] = ih_v[sl] * 4 + 3
            return carry2
        lax.fori_loop(0, HIST_IDX // 16, hidx_body, 0, unroll=4)

        cps = [
            pltpu.async_copy(tab4.at[iu_v], ru_v, sem),
            pltpu.async_copy(tab4.at[ip_v], rp_v, sem),
            pltpu.async_copy(tab4.at[ic_v], rc_v, sem),
            pltpu.async_copy(tab4.at[ii_v], ri_v, sem),
        ]
        for k in range(NGATHER):
            cps.append(pltpu.async_copy(
                tab4.at[ih_v.at[pl.ds(k * GSZ, GSZ)]],
                rh_v.at[pl.ds(k * GSZ, GSZ)], sem))
        for cp in cps:
            cp.wait()

        def row_body(b, carry2):
            acc = rh_v[b * HIST, :]
            for j in range(1, HIST):
                acc = acc + rh_v[b * HIST + j, :]
            hs_v[b, :] = acc
            return carry2
        lax.fori_loop(0, CB, row_body, 0, unroll=False)

        pltpu.sync_copy(ru_v, out_u.at[pl.ds(base, CB)])
        pltpu.sync_copy(rp_v, out_p.at[pl.ds(base, CB)])
        pltpu.sync_copy(rc_v, out_c.at[pl.ds(base, CB)])
        pltpu.sync_copy(ri_v, out_i.at[pl.ds(base, CB)])
        pltpu.sync_copy(hs_v, out_h.at[pl.ds(base, CB)])
        return carry

    lax.fori_loop(0, NCHUNK, chunk_body, 0, unroll=False)


@functools.cache
def _sc_gather():
    return pl.kernel(
        _sc_gather_body,
        out_type=[jax.ShapeDtypeStruct((B, D), jnp.float32)] * 5,
        mesh=plsc.VectorSubcoreMesh(core_axis_name="c", subcore_axis_name="s",
                                    num_cores=NC, num_subcores=NS),
        compiler_params=pltpu.CompilerParams(use_tc_tiling_on_sc=False),
        scratch_types=[
        pltpu.VMEM((CB,), jnp.int32),
        pltpu.VMEM((CB,), jnp.int32),
        pltpu.VMEM((CB,), jnp.int32),
        pltpu.VMEM((CB,), jnp.int32),
        pltpu.VMEM((HIST_IDX,), jnp.int32),
        pltpu.VMEM((CB, D), jnp.float32),
        pltpu.VMEM((CB, D), jnp.float32),
        pltpu.VMEM((CB, D), jnp.float32),
        pltpu.VMEM((CB, D), jnp.float32),
        pltpu.VMEM((HIST_IDX, D), jnp.float32),
            pltpu.VMEM((CB, D), jnp.float32),
            pltpu.SemaphoreType.DMA,
        ],
    )


RB = 8192


def _tc_repack_body(u_ref, p_ref, c_ref, i_ref, o_ref):
    x4 = jnp.concatenate([u_ref[:], p_ref[:], c_ref[:], i_ref[:]], axis=0)
    eye = (jax.lax.broadcasted_iota(jnp.int32, (4 * D, 4 * D), 0)
           == jax.lax.broadcasted_iota(jnp.int32, (4 * D, 4 * D), 1)
           ).astype(jnp.float32)
    o_ref[:] = jax.lax.dot_general(x4, eye, (((0,), (0,)), ((), ())),
                                   preferred_element_type=jnp.float32)


def _tc_repack(user_tt, prov_tt, city_tt, item_tt):
    grid = (pl.cdiv(ITEM_ROWS + 1, RB),)
    col = pl.BlockSpec((D, RB), lambda g: (0, g))
    rowb = pl.BlockSpec((RB, 4 * D), lambda g: (g, 0))
    return pl.pallas_call(
        _tc_repack_body,
        grid=grid,
        in_specs=[col, col, col, col],
        out_specs=rowb,
        out_shape=jax.ShapeDtypeStruct((ITEM_ROWS, 4 * D), jnp.float32),
    )(user_tt, prov_tt, city_tt, item_tt)


BM = 2048


def _tc_mlp_body(u_ref, p_ref, c_ref, i_ref, h_ref, x_ref,
                 w1u, w1p, w1c, w1h, w1x, b1_ref,
                 w2_ref, b2_ref, w3_ref, b3_ref, o_ref):
    f32 = jnp.float32
    h1 = (jnp.dot(u_ref[:], w1u[:], preferred_element_type=f32)
          + jnp.dot(p_ref[:], w1p[:], preferred_element_type=f32)
          + jnp.dot(c_ref[:], w1c[:], preferred_element_type=f32)
          + jnp.dot(h_ref[:] * (1.0 / HIST), w1h[:], preferred_element_type=f32)
          + jnp.dot(x_ref[:], w1x[:], preferred_element_type=f32)
          + b1_ref[:])
    h2 = jnp.dot(h1, w2_ref[:], preferred_element_type=f32) + b2_ref[:]
    u = jnp.dot(h2, w3_ref[:], preferred_element_type=f32) + b3_ref[:]
    logits = jnp.sum(u * i_ref[:], axis=1, keepdims=True)
    o_ref[:] = jax.nn.sigmoid(logits)


def _tc_mlp(user_e, prov_e, city_e, item_e, hist_s, cont,
            w1u, w1p, w1c, w1h, w1x, b1, w2, b2, w3, b3):
    row = lambda bm, w: pl.BlockSpec((bm, w), lambda i: (i, 0))
    full = lambda s: pl.BlockSpec(s, lambda i: (0, 0))
    return pl.pallas_call(
        _tc_mlp_body,
        grid=(B // BM,),
        in_specs=[
            row(BM, D), row(BM, D), row(BM, D), row(BM, D), row(BM, D),
            row(BM, 19),
            full((D, 128)), full((D, 128)), full((D, 128)), full((D, 128)),
            full((19, 128)), full((1, 128)),
            full((128, 32)), full((1, 32)),
            full((32, 16)), full((1, 16)),
        ],
        out_specs=pl.BlockSpec((BM, 1), lambda i: (i, 0)),
        out_shape=jax.ShapeDtypeStruct((B, 1), jnp.float32),
    )(user_e, prov_e, city_e, item_e, hist_s, cont,
      w1u, w1p, w1c, w1h, w1x, b1, w2, b2, w3, b3)


def kernel(disc, cont, itemId, item_table, user_table, city_table, prov_table,
           W1, b1, W2, b2, W3, b3):
    idx_u = disc[:, 0]
    idx_p = disc[:, 1]
    idx_c = disc[:, 2]
    idx_h = disc[:, 3:].reshape(-1)

    packed = _tc_repack(
        user_table.T, prov_table.T, city_table.T, item_table.T)
    packed4 = packed.reshape(4 * ITEM_ROWS, D)

    user_e, prov_e, city_e, item_e, hist_s = _sc_gather()(
        idx_u, idx_p, idx_c, itemId, idx_h, packed4)

    out = _tc_mlp(
        user_e, prov_e, city_e, item_e, hist_s, cont,
        W1[0:16], W1[16:32], W1[32:48], W1[48:64], W1[64:83],
        b1.reshape(1, 128), W2, b2.reshape(1, 32), W3, b3.reshape(1, 16))
    return out.reshape(B)


# speedup vs baseline: 2.7985x; 1.0310x over previous
import functools

import jax
import jax.numpy as jnp
from jax import lax
from jax.experimental import pallas as pl
from jax.experimental.pallas import tpu as pltpu
from jax.experimental.pallas import tpu_sc as plsc

B = 16384
HIST = 50
D = 16
ITEM_ROWS = 1000000

NC = 2
NS = 16
NW = NC * NS
BPW = B // NW
CB = 64
NCHUNK = BPW // CB
HIST_IDX = CB * HIST
GSZ = 128
NGATHER = HIST_IDX // GSZ


def _sc_gather_body(idx_u_hbm, idx_p_hbm, idx_c_hbm, idx_i_hbm, idx_h_hbm,
                    tab4,
                    out_u, out_p, out_c, out_i, out_h,
                    iu_v, ip_v, ic_v, ii_v, ih_v,
                    ru_v, rp_v, rc_v, ri_v, rh_v, hs_v, sem):
    wid = lax.axis_index("s") * NC + lax.axis_index("c")

    def chunk_body(c, carry):
        base = wid * BPW + c * CB
        pltpu.sync_copy(idx_u_hbm.at[pl.ds(base, CB)], iu_v)
        pltpu.sync_copy(idx_p_hbm.at[pl.ds(base, CB)], ip_v)
        pltpu.sync_copy(idx_c_hbm.at[pl.ds(base, CB)], ic_v)
        pltpu.sync_copy(idx_i_hbm.at[pl.ds(base, CB)], ii_v)
        pltpu.sync_copy(idx_h_hbm.at[pl.ds(base * HIST, HIST_IDX)], ih_v)

        for s_ in range(CB // 16):
            sl = pl.ds(s_ * 16, 16)
            iu_v[sl] = iu_v[sl] * 4
            ip_v[sl] = ip_v[sl] * 4 + 1
            ic_v[sl] = ic_v[sl] * 4 + 2
            ii_v[sl] = ii_v[sl] * 4 + 3

        def hidx_body(s_, carry2):
            sl = pl.ds(s_ * 16, 16)
            ih_v[sl] = ih_v[sl] * 4 + 3
            return carry2
        lax.fori_loop(0, HIST_IDX // 16, hidx_body, 0, unroll=4)

        cps = [
            pltpu.async_copy(tab4.at[iu_v], ru_v, sem),
            pltpu.async_copy(tab4.at[ip_v], rp_v, sem),
            pltpu.async_copy(tab4.at[ic_v], rc_v, sem),
            pltpu.async_copy(tab4.at[ii_v], ri_v, sem),
        ]
        for k in range(NGATHER):
            cps.append(pltpu.async_copy(
                tab4.at[ih_v.at[pl.ds(k * GSZ, GSZ)]],
                rh_v.at[pl.ds(k * GSZ, GSZ)], sem))
        for cp in cps:
            cp.wait()

        def row_body(b, carry2):
            acc = rh_v[b * HIST, :]
            for j in range(1, HIST):
                acc = acc + rh_v[b * HIST + j, :]
            hs_v[b, :] = acc
            return carry2
        lax.fori_loop(0, CB, row_body, 0, unroll=False)

        pltpu.sync_copy(ru_v, out_u.at[pl.ds(base, CB)])
        pltpu.sync_copy(rp_v, out_p.at[pl.ds(base, CB)])
        pltpu.sync_copy(rc_v, out_c.at[pl.ds(base, CB)])
        pltpu.sync_copy(ri_v, out_i.at[pl.ds(base, CB)])
        pltpu.sync_copy(hs_v, out_h.at[pl.ds(base, CB)])
        return carry

    lax.fori_loop(0, NCHUNK, chunk_body, 0, unroll=False)


@functools.cache
def _sc_gather():
    return pl.kernel(
        _sc_gather_body,
        out_type=[jax.ShapeDtypeStruct((B, D), jnp.float32)] * 5,
        mesh=plsc.VectorSubcoreMesh(core_axis_name="c", subcore_axis_name="s",
                                    num_cores=NC, num_subcores=NS),
        compiler_params=pltpu.CompilerParams(use_tc_tiling_on_sc=False),
        scratch_types=[
        pltpu.VMEM((CB,), jnp.int32),
        pltpu.VMEM((CB,), jnp.int32),
        pltpu.VMEM((CB,), jnp.int32),
        pltpu.VMEM((CB,), jnp.int32),
        pltpu.VMEM((HIST_IDX,), jnp.int32),
        pltpu.VMEM((CB, D), jnp.float32),
        pltpu.VMEM((CB, D), jnp.float32),
        pltpu.VMEM((CB, D), jnp.float32),
        pltpu.VMEM((CB, D), jnp.float32),
        pltpu.VMEM((HIST_IDX, D), jnp.float32),
            pltpu.VMEM((CB, D), jnp.float32),
            pltpu.SemaphoreType.DMA,
        ],
    )


RB = 16384


def _tc_repack_body(u_ref, p_ref, c_ref, i_ref, o_ref):
    x4 = jnp.concatenate([u_ref[:], p_ref[:], c_ref[:], i_ref[:]], axis=0)
    eye = (jax.lax.broadcasted_iota(jnp.int32, (4 * D, 4 * D), 0)
           == jax.lax.broadcasted_iota(jnp.int32, (4 * D, 4 * D), 1)
           ).astype(jnp.float32)
    o_ref[:] = jax.lax.dot_general(x4, eye, (((0,), (0,)), ((), ())),
                                   preferred_element_type=jnp.float32)


def _tc_repack(user_tt, prov_tt, city_tt, item_tt):
    grid = (pl.cdiv(ITEM_ROWS + 1, RB),)
    col = pl.BlockSpec((D, RB), lambda g: (0, g))
    rowb = pl.BlockSpec((RB, 4 * D), lambda g: (g, 0))
    return pl.pallas_call(
        _tc_repack_body,
        grid=grid,
        in_specs=[col, col, col, col],
        out_specs=rowb,
        out_shape=jax.ShapeDtypeStruct((ITEM_ROWS, 4 * D), jnp.float32),
    )(user_tt, prov_tt, city_tt, item_tt)


BM = 2048


def _tc_mlp_body(u_ref, p_ref, c_ref, i_ref, h_ref, x_ref,
                 w1u, w1p, w1c, w1h, w1x, b1_ref,
                 w2_ref, b2_ref, w3_ref, b3_ref, o_ref):
    f32 = jnp.float32
    h1 = (jnp.dot(u_ref[:], w1u[:], preferred_element_type=f32)
          + jnp.dot(p_ref[:], w1p[:], preferred_element_type=f32)
          + jnp.dot(c_ref[:], w1c[:], preferred_element_type=f32)
          + jnp.dot(h_ref[:] * (1.0 / HIST), w1h[:], preferred_element_type=f32)
          + jnp.dot(x_ref[:], w1x[:], preferred_element_type=f32)
          + b1_ref[:])
    h2 = jnp.dot(h1, w2_ref[:], preferred_element_type=f32) + b2_ref[:]
    u = jnp.dot(h2, w3_ref[:], preferred_element_type=f32) + b3_ref[:]
    logits = jnp.sum(u * i_ref[:], axis=1, keepdims=True)
    o_ref[:] = jax.nn.sigmoid(logits)


def _tc_mlp(user_e, prov_e, city_e, item_e, hist_s, cont,
            w1u, w1p, w1c, w1h, w1x, b1, w2, b2, w3, b3):
    row = lambda bm, w: pl.BlockSpec((bm, w), lambda i: (i, 0))
    full = lambda s: pl.BlockSpec(s, lambda i: (0, 0))
    return pl.pallas_call(
        _tc_mlp_body,
        grid=(B // BM,),
        in_specs=[
            row(BM, D), row(BM, D), row(BM, D), row(BM, D), row(BM, D),
            row(BM, 19),
            full((D, 128)), full((D, 128)), full((D, 128)), full((D, 128)),
            full((19, 128)), full((1, 128)),
            full((128, 32)), full((1, 32)),
            full((32, 16)), full((1, 16)),
        ],
        out_specs=pl.BlockSpec((BM, 1), lambda i: (i, 0)),
        out_shape=jax.ShapeDtypeStruct((B, 1), jnp.float32),
    )(user_e, prov_e, city_e, item_e, hist_s, cont,
      w1u, w1p, w1c, w1h, w1x, b1, w2, b2, w3, b3)


def kernel(disc, cont, itemId, item_table, user_table, city_table, prov_table,
           W1, b1, W2, b2, W3, b3):
    idx_u = disc[:, 0]
    idx_p = disc[:, 1]
    idx_c = disc[:, 2]
    idx_h = disc[:, 3:].reshape(-1)

    packed = _tc_repack(
        user_table.T, prov_table.T, city_table.T, item_table.T)
    packed4 = packed.reshape(4 * ITEM_ROWS, D)

    user_e, prov_e, city_e, item_e, hist_s = _sc_gather()(
        idx_u, idx_p, idx_c, itemId, idx_h, packed4)

    out = _tc_mlp(
        user_e, prov_e, city_e, item_e, hist_s, cont,
        W1[0:16], W1[16:32], W1[32:48], W1[48:64], W1[64:83],
        b1.reshape(1, 128), W2, b2.reshape(1, 32), W3, b3.reshape(1, 16))
    return out.reshape(B)


# speedup vs baseline: 2.8146x; 1.0057x over previous
import functools

import jax
import jax.numpy as jnp
from jax import lax
from jax.experimental import pallas as pl
from jax.experimental.pallas import tpu as pltpu
from jax.experimental.pallas import tpu_sc as plsc

B = 16384
HIST = 50
D = 16
ITEM_ROWS = 1000000

NC = 2
NS = 16
NW = NC * NS
BPW = B // NW
CB = 64
NCHUNK = BPW // CB
HIST_IDX = CB * HIST
GSZ = 128
NGATHER = HIST_IDX // GSZ


def _sc_gather_body(idx_u_hbm, idx_p_hbm, idx_c_hbm, idx_i_hbm, idx_h_hbm,
                    tab4,
                    out_u, out_p, out_c, out_i, out_h,
                    iu_v, ip_v, ic_v, ii_v, ih_v,
                    ru_v, rp_v, rc_v, ri_v, rh_v, hs_v, sem):
    wid = lax.axis_index("s") * NC + lax.axis_index("c")

    def chunk_body(c, carry):
        base = wid * BPW + c * CB
        pltpu.sync_copy(idx_u_hbm.at[pl.ds(base, CB)], iu_v)
        pltpu.sync_copy(idx_p_hbm.at[pl.ds(base, CB)], ip_v)
        pltpu.sync_copy(idx_c_hbm.at[pl.ds(base, CB)], ic_v)
        pltpu.sync_copy(idx_i_hbm.at[pl.ds(base, CB)], ii_v)
        pltpu.sync_copy(idx_h_hbm.at[pl.ds(base * HIST, HIST_IDX)], ih_v)

        for s_ in range(CB // 16):
            sl = pl.ds(s_ * 16, 16)
            iu_v[sl] = iu_v[sl] * 4
            ip_v[sl] = ip_v[sl] * 4 + 1
            ic_v[sl] = ic_v[sl] * 4 + 2
            ii_v[sl] = ii_v[sl] * 4 + 3

        def hidx_body(s_, carry2):
            sl = pl.ds(s_ * 16, 16)
            ih_v[sl] = ih_v[sl] * 4 + 3
            return carry2
        lax.fori_loop(0, HIST_IDX // 16, hidx_body, 0, unroll=4)

        cps = [
            pltpu.async_copy(tab4.at[iu_v], ru_v, sem),
            pltpu.async_copy(tab4.at[ip_v], rp_v, sem),
            pltpu.async_copy(tab4.at[ic_v], rc_v, sem),
            pltpu.async_copy(tab4.at[ii_v], ri_v, sem),
        ]
        for k in range(NGATHER):
            cps.append(pltpu.async_copy(
                tab4.at[ih_v.at[pl.ds(k * GSZ, GSZ)]],
                rh_v.at[pl.ds(k * GSZ, GSZ)], sem))
        for cp in cps:
            cp.wait()

        def row_body(b, carry2):
            acc = rh_v[b * HIST, :]
            for j in range(1, HIST):
                acc = acc + rh_v[b * HIST + j, :]
            hs_v[b, :] = acc
            return carry2
        lax.fori_loop(0, CB, row_body, 0, unroll=False)

        pltpu.sync_copy(ru_v, out_u.at[pl.ds(base, CB)])
        pltpu.sync_copy(rp_v, out_p.at[pl.ds(base, CB)])
        pltpu.sync_copy(rc_v, out_c.at[pl.ds(base, CB)])
        pltpu.sync_copy(ri_v, out_i.at[pl.ds(base, CB)])
        pltpu.sync_copy(hs_v, out_h.at[pl.ds(base, CB)])
        return carry

    lax.fori_loop(0, NCHUNK, chunk_body, 0, unroll=False)


@functools.cache
def _sc_gather():
    return pl.kernel(
        _sc_gather_body,
        out_type=[jax.ShapeDtypeStruct((B, D), jnp.float32)] * 5,
        mesh=plsc.VectorSubcoreMesh(core_axis_name="c", subcore_axis_name="s",
                                    num_cores=NC, num_subcores=NS),
        compiler_params=pltpu.CompilerParams(use_tc_tiling_on_sc=False),
        scratch_types=[
        pltpu.VMEM((CB,), jnp.int32),
        pltpu.VMEM((CB,), jnp.int32),
        pltpu.VMEM((CB,), jnp.int32),
        pltpu.VMEM((CB,), jnp.int32),
        pltpu.VMEM((HIST_IDX,), jnp.int32),
        pltpu.VMEM((CB, D), jnp.float32),
        pltpu.VMEM((CB, D), jnp.float32),
        pltpu.VMEM((CB, D), jnp.float32),
        pltpu.VMEM((CB, D), jnp.float32),
        pltpu.VMEM((HIST_IDX, D), jnp.float32),
            pltpu.VMEM((CB, D), jnp.float32),
            pltpu.SemaphoreType.DMA,
        ],
    )


RB = 32768


def _tc_repack_body(u_ref, p_ref, c_ref, i_ref, o_ref):
    x4 = jnp.concatenate([u_ref[:], p_ref[:], c_ref[:], i_ref[:]], axis=0)
    eye = (jax.lax.broadcasted_iota(jnp.int32, (4 * D, 4 * D), 0)
           == jax.lax.broadcasted_iota(jnp.int32, (4 * D, 4 * D), 1)
           ).astype(jnp.float32)
    o_ref[:] = jax.lax.dot_general(x4, eye, (((0,), (0,)), ((), ())),
                                   preferred_element_type=jnp.float32)


def _tc_repack(user_tt, prov_tt, city_tt, item_tt):
    grid = (pl.cdiv(ITEM_ROWS + 1, RB),)
    col = pl.BlockSpec((D, RB), lambda g: (0, g))
    rowb = pl.BlockSpec((RB, 4 * D), lambda g: (g, 0))
    return pl.pallas_call(
        _tc_repack_body,
        grid=grid,
        in_specs=[col, col, col, col],
        out_specs=rowb,
        out_shape=jax.ShapeDtypeStruct((ITEM_ROWS, 4 * D), jnp.float32),
    )(user_tt, prov_tt, city_tt, item_tt)


BM = 2048


def _tc_mlp_body(u_ref, p_ref, c_ref, i_ref, h_ref, x_ref,
                 w1u, w1p, w1c, w1h, w1x, b1_ref,
                 w2_ref, b2_ref, w3_ref, b3_ref, o_ref):
    f32 = jnp.float32
    h1 = (jnp.dot(u_ref[:], w1u[:], preferred_element_type=f32)
          + jnp.dot(p_ref[:], w1p[:], preferred_element_type=f32)
          + jnp.dot(c_ref[:], w1c[:], preferred_element_type=f32)
          + jnp.dot(h_ref[:] * (1.0 / HIST), w1h[:], preferred_element_type=f32)
          + jnp.dot(x_ref[:], w1x[:], preferred_element_type=f32)
          + b1_ref[:])
    h2 = jnp.dot(h1, w2_ref[:], preferred_element_type=f32) + b2_ref[:]
    u = jnp.dot(h2, w3_ref[:], preferred_element_type=f32) + b3_ref[:]
    logits = jnp.sum(u * i_ref[:], axis=1, keepdims=True)
    o_ref[:] = jax.nn.sigmoid(logits)


def _tc_mlp(user_e, prov_e, city_e, item_e, hist_s, cont,
            w1u, w1p, w1c, w1h, w1x, b1, w2, b2, w3, b3):
    row = lambda bm, w: pl.BlockSpec((bm, w), lambda i: (i, 0))
    full = lambda s: pl.BlockSpec(s, lambda i: (0, 0))
    return pl.pallas_call(
        _tc_mlp_body,
        grid=(B // BM,),
        in_specs=[
            row(BM, D), row(BM, D), row(BM, D), row(BM, D), row(BM, D),
            row(BM, 19),
            full((D, 128)), full((D, 128)), full((D, 128)), full((D, 128)),
            full((19, 128)), full((1, 128)),
            full((128, 32)), full((1, 32)),
            full((32, 16)), full((1, 16)),
        ],
        out_specs=pl.BlockSpec((BM, 1), lambda i: (i, 0)),
        out_shape=jax.ShapeDtypeStruct((B, 1), jnp.float32),
    )(user_e, prov_e, city_e, item_e, hist_s, cont,
      w1u, w1p, w1c, w1h, w1x, b1, w2, b2, w3, b3)


def kernel(disc, cont, itemId, item_table, user_table, city_table, prov_table,
           W1, b1, W2, b2, W3, b3):
    idx_u = disc[:, 0]
    idx_p = disc[:, 1]
    idx_c = disc[:, 2]
    idx_h = disc[:, 3:].reshape(-1)

    packed = _tc_repack(
        user_table.T, prov_table.T, city_table.T, item_table.T)
    packed4 = packed.reshape(4 * ITEM_ROWS, D)

    user_e, prov_e, city_e, item_e, hist_s = _sc_gather()(
        idx_u, idx_p, idx_c, itemId, idx_h, packed4)

    out = _tc_mlp(
        user_e, prov_e, city_e, item_e, hist_s, cont,
        W1[0:16], W1[16:32], W1[32:48], W1[48:64], W1[64:83],
        b1.reshape(1, 128), W2, b2.reshape(1, 32), W3, b3.reshape(1, 16))
    return out.reshape(B)


# speedup vs baseline: 2.8220x; 1.0026x over previous
import functools

import jax
import jax.numpy as jnp
from jax import lax
from jax.experimental import pallas as pl
from jax.experimental.pallas import tpu as pltpu
from jax.experimental.pallas import tpu_sc as plsc

B = 16384
HIST = 50
D = 16
ITEM_ROWS = 1000000

NC = 2
NS = 16
NW = NC * NS
BPW = B // NW
CB = 64
NCHUNK = BPW // CB
HIST_IDX = CB * HIST
GSZ = 128
NGATHER = HIST_IDX // GSZ


def _sc_gather_body(idx_u_hbm, idx_p_hbm, idx_c_hbm, idx_i_hbm, idx_h_hbm,
                    tab4,
                    out_u, out_p, out_c, out_i, out_h,
                    iu_v, ip_v, ic_v, ii_v, ih_v,
                    ru_v, rp_v, rc_v, ri_v, rh_v, hs_v, sem):
    wid = lax.axis_index("s") * NC + lax.axis_index("c")

    def chunk_body(c, carry):
        base = wid * BPW + c * CB
        pltpu.sync_copy(idx_u_hbm.at[pl.ds(base, CB)], iu_v)
        pltpu.sync_copy(idx_p_hbm.at[pl.ds(base, CB)], ip_v)
        pltpu.sync_copy(idx_c_hbm.at[pl.ds(base, CB)], ic_v)
        pltpu.sync_copy(idx_i_hbm.at[pl.ds(base, CB)], ii_v)
        pltpu.sync_copy(idx_h_hbm.at[pl.ds(base * HIST, HIST_IDX)], ih_v)

        for s_ in range(CB // 16):
            sl = pl.ds(s_ * 16, 16)
            iu_v[sl] = iu_v[sl] * 4
            ip_v[sl] = ip_v[sl] * 4 + 1
            ic_v[sl] = ic_v[sl] * 4 + 2
            ii_v[sl] = ii_v[sl] * 4 + 3

        def hidx_body(s_, carry2):
            sl = pl.ds(s_ * 16, 16)
            ih_v[sl] = ih_v[sl] * 4 + 3
            return carry2
        lax.fori_loop(0, HIST_IDX // 16, hidx_body, 0, unroll=4)

        cps = [
            pltpu.async_copy(tab4.at[iu_v], ru_v, sem),
            pltpu.async_copy(tab4.at[ip_v], rp_v, sem),
            pltpu.async_copy(tab4.at[ic_v], rc_v, sem),
            pltpu.async_copy(tab4.at[ii_v], ri_v, sem),
        ]
        for k in range(NGATHER):
            cps.append(pltpu.async_copy(
                tab4.at[ih_v.at[pl.ds(k * GSZ, GSZ)]],
                rh_v.at[pl.ds(k * GSZ, GSZ)], sem))
        for cp in cps:
            cp.wait()

        def row_body(b, carry2):
            acc = rh_v[b * HIST, :]
            for j in range(1, HIST):
                acc = acc + rh_v[b * HIST + j, :]
            hs_v[b, :] = acc
            return carry2
        lax.fori_loop(0, CB, row_body, 0, unroll=False)

        pltpu.sync_copy(ru_v, out_u.at[pl.ds(base, CB)])
        pltpu.sync_copy(rp_v, out_p.at[pl.ds(base, CB)])
        pltpu.sync_copy(rc_v, out_c.at[pl.ds(base, CB)])
        pltpu.sync_copy(ri_v, out_i.at[pl.ds(base, CB)])
        pltpu.sync_copy(hs_v, out_h.at[pl.ds(base, CB)])
        return carry

    lax.fori_loop(0, NCHUNK, chunk_body, 0, unroll=False)


@functools.cache
def _sc_gather():
    return pl.kernel(
        _sc_gather_body,
        out_type=[jax.ShapeDtypeStruct((B, D), jnp.float32)] * 5,
        mesh=plsc.VectorSubcoreMesh(core_axis_name="c", subcore_axis_name="s",
                                    num_cores=NC, num_subcores=NS),
        compiler_params=pltpu.CompilerParams(use_tc_tiling_on_sc=False),
        scratch_types=[
        pltpu.VMEM((CB,), jnp.int32),
        pltpu.VMEM((CB,), jnp.int32),
        pltpu.VMEM((CB,), jnp.int32),
        pltpu.VMEM((CB,), jnp.int32),
        pltpu.VMEM((HIST_IDX,), jnp.int32),
        pltpu.VMEM((CB, D), jnp.float32),
        pltpu.VMEM((CB, D), jnp.float32),
        pltpu.VMEM((CB, D), jnp.float32),
        pltpu.VMEM((CB, D), jnp.float32),
        pltpu.VMEM((HIST_IDX, D), jnp.float32),
            pltpu.VMEM((CB, D), jnp.float32),
            pltpu.SemaphoreType.DMA,
        ],
    )


RB = 32768


def _tc_repack_body(u_ref, p_ref, c_ref, i_ref, o_ref):
    x4 = jnp.concatenate([u_ref[:], p_ref[:], c_ref[:], i_ref[:]], axis=0)
    eye = (jax.lax.broadcasted_iota(jnp.int32, (4 * D, 4 * D), 0)
           == jax.lax.broadcasted_iota(jnp.int32, (4 * D, 4 * D), 1)
           ).astype(jnp.float32)
    o_ref[:] = jax.lax.dot_general(x4, eye, (((0,), (0,)), ((), ())),
                                   preferred_element_type=jnp.float32)


def _tc_repack(user_tt, prov_tt, city_tt, item_tt):
    grid = (pl.cdiv(ITEM_ROWS + 1, RB),)
    col = pl.BlockSpec((D, RB), lambda g: (0, g))
    rowb = pl.BlockSpec((RB, 4 * D), lambda g: (g, 0))
    return pl.pallas_call(
        _tc_repack_body,
        grid=grid,
        in_specs=[col, col, col, col],
        out_specs=rowb,
        out_shape=jax.ShapeDtypeStruct((ITEM_ROWS, 4 * D), jnp.float32),
    )(user_tt, prov_tt, city_tt, item_tt)


BM = 4096


def _tc_mlp_body(u_ref, p_ref, c_ref, i_ref, h_ref, x_ref,
                 w1u, w1p, w1c, w1h, w1x, b1_ref,
                 w2_ref, b2_ref, w3_ref, b3_ref, o_ref):
    f32 = jnp.float32
    h1 = (jnp.dot(u_ref[:], w1u[:], preferred_element_type=f32)
          + jnp.dot(p_ref[:], w1p[:], preferred_element_type=f32)
          + jnp.dot(c_ref[:], w1c[:], preferred_element_type=f32)
          + jnp.dot(h_ref[:] * (1.0 / HIST), w1h[:], preferred_element_type=f32)
          + jax.lax.dot_general(x_ref[:], w1x[:], (((0,), (0,)), ((), ())),
                                preferred_element_type=f32)
          + b1_ref[:])
    h2 = jnp.dot(h1, w2_ref[:], preferred_element_type=f32) + b2_ref[:]
    u = jnp.dot(h2, w3_ref[:], preferred_element_type=f32) + b3_ref[:]
    logits = jnp.sum(u * i_ref[:], axis=1, keepdims=True)
    o_ref[:] = jax.nn.sigmoid(logits)


def _tc_mlp(user_e, prov_e, city_e, item_e, hist_s, cont,
            w1u, w1p, w1c, w1h, w1x, b1, w2, b2, w3, b3):
    row = lambda bm, w: pl.BlockSpec((bm, w), lambda i: (i, 0))
    full = lambda s: pl.BlockSpec(s, lambda i: (0, 0))
    return pl.pallas_call(
        _tc_mlp_body,
        grid=(B // BM,),
        in_specs=[
            row(BM, D), row(BM, D), row(BM, D), row(BM, D), row(BM, D),
            pl.BlockSpec((19, BM), lambda i: (0, i)),
            full((D, 128)), full((D, 128)), full((D, 128)), full((D, 128)),
            full((19, 128)), full((1, 128)),
            full((128, 32)), full((1, 32)),
            full((32, 16)), full((1, 16)),
        ],
        out_specs=pl.BlockSpec((BM, 1), lambda i: (i, 0)),
        out_shape=jax.ShapeDtypeStruct((B, 1), jnp.float32),
    )(user_e, prov_e, city_e, item_e, hist_s, cont,
      w1u, w1p, w1c, w1h, w1x, b1, w2, b2, w3, b3)


def kernel(disc, cont, itemId, item_table, user_table, city_table, prov_table,
           W1, b1, W2, b2, W3, b3):
    idx_u = disc[:, 0]
    idx_p = disc[:, 1]
    idx_c = disc[:, 2]
    idx_h = disc[:, 3:].reshape(-1)

    packed = _tc_repack(
        user_table.T, prov_table.T, city_table.T, item_table.T)
    packed4 = packed.reshape(4 * ITEM_ROWS, D)

    user_e, prov_e, city_e, item_e, hist_s = _sc_gather()(
        idx_u, idx_p, idx_c, itemId, idx_h, packed4)

    out = _tc_mlp(
        user_e, prov_e, city_e, item_e, hist_s, cont.T,
        W1[0:16], W1[16:32], W1[32:48], W1[48:64], W1[64:83],
        b1.reshape(1, 128), W2, b2.reshape(1, 32), W3, b3.reshape(1, 16))
    return out.reshape(B)
